# Initial kernel scaffold; baseline (speedup 1.0000x reference)
import jax
import jax.numpy as jnp
from jax.experimental import pallas as pl


def kernel(x, edge_index, pairwise_indices, W_msg, b_msg, W_upd, b_upd, W_p1, b_p1, gamma, beta, W_p2, b_p2):
    raise NotImplementedError("write your pallas kernel here")


# speedup vs baseline: 2.9170x; 2.9170x over previous
import functools

import jax
import jax.numpy as jnp
from jax import lax
from jax.experimental import pallas as pl
from jax.experimental.pallas import tpu as pltpu
from jax.experimental.pallas import tpu_sc as plsc

_N = 10000
_E = 320000
_P = 500000
_H = 128
_HID = 32

_NW = 32
_R = 313
_RROWS = _R + 1
_ACCW = _RROWS * _H
_DEGW = 336
_C = 512
_NCHUNK = _E // _C
_CAP = 1024
_DRAIN_T = 384
_NEG = -3.0e38
_POS = 3.0e38

_PB = 16384
_PPAD = _PB * _NW
_PC = 1024


def _tc1_body(x_ref, w1_ref, w2_ref, b_ref, a_ref, b_out_ref):
    xv = x_ref[...]
    a_ref[...] = jnp.dot(xv, w1_ref[...], preferred_element_type=jnp.float32)
    b_out_ref[...] = (
        jnp.dot(xv, w2_ref[...], preferred_element_type=jnp.float32) + b_ref[...]
    )


def _tc1(x, w1, w2, bmsg):
    blk = 1000
    grid = _N // blk
    return pl.pallas_call(
        _tc1_body,
        grid=(grid,),
        in_specs=[
            pl.BlockSpec((blk, _H), lambda i: (i, 0)),
            pl.BlockSpec((_H, _H), lambda i: (0, 0)),
            pl.BlockSpec((_H, _H), lambda i: (0, 0)),
            pl.BlockSpec((1, _H), lambda i: (0, 0)),
        ],
        out_specs=[
            pl.BlockSpec((blk, _H), lambda i: (i, 0)),
            pl.BlockSpec((blk, _H), lambda i: (i, 0)),
        ],
        out_shape=[
            jax.ShapeDtypeStruct((_N, _H), jnp.float32),
            jax.ShapeDtypeStruct((_N, _H), jnp.float32),
        ],
    )(x, w1, w2, bmsg)


def _edge_body(a_hbm, b_hbm, src_hbm, dst_hbm,
               s_out, mx_out, mn_out, deg_out,
               s_acc, mx_acc, mn_acc, deg_acc,
               srcbuf, dstbuf, csrc, cdst, cglb,
               stage_a, stage_b, sem_a, sem_b):
    w = lax.axis_index("s") * 2 + lax.axis_index("c")
    base = w * _R
    iota = lax.iota(jnp.int32, 16)
    onehot = jnp.where(iota == 0, 1.0, 0.0).astype(jnp.float32)
    zero16f = jnp.zeros((16,), jnp.float32)

    def init_body(i, _):
        sl = pl.ds(i * 16, 16)
        s_acc[sl] = zero16f
        mx_acc[sl] = jnp.full((16,), _NEG, jnp.float32)
        mn_acc[sl] = jnp.full((16,), _POS, jnp.float32)
        return 0
    lax.fori_loop(0, _ACCW // 16, init_body, 0)

    def initd_body(i, _):
        deg_acc[pl.ds(i * 16, 16)] = zero16f
        return 0
    lax.fori_loop(0, _DEGW // 16, initd_body, 0)

    def drain(cnt):
        csrc[pl.ds(cnt, 16)] = jnp.zeros((16,), jnp.int32)
        cglb[pl.ds(cnt, 16)] = jnp.zeros((16,), jnp.int32)
        cdst[pl.ds(cnt, 16)] = jnp.full((16,), _R, jnp.int32)
        nvec = (cnt + 15) // 16

        def gbody(j, _):
            sl = pl.ds(j * 16, 16)
            cpa = pltpu.async_copy(a_hbm.at[csrc.at[sl]], stage_a, sem_a)
            cpb = pltpu.async_copy(b_hbm.at[cglb.at[sl]], stage_b, sem_b)
            cpa.wait()
            cpb.wait()
            dlv = cdst[sl]

            def edge_body(k, _):
                dk = jnp.sum(jnp.where(iota == k, dlv, 0))
                bofs = dk * _H
                for r in range(_H // 16):
                    asl = stage_a.at[k][pl.ds(r * 16, 16)]
                    bsl = stage_b.at[k][pl.ds(r * 16, 16)]
                    m = jnp.maximum(asl + bsl, 0.0)
                    accsl = pl.ds(bofs + r * 16, 16)
                    plsc.addupdate(s_acc.at[accsl], m)
                    mx_acc[accsl] = jnp.maximum(mx_acc[accsl], m)
                    mn_acc[accsl] = jnp.minimum(mn_acc[accsl], m)
                plsc.addupdate(deg_acc.at[pl.ds(dk, 16)], onehot)
                return 0
            lax.fori_loop(0, 16, edge_body, 0)
            return 0
        lax.fori_loop(0, nvec, gbody, 0)
        return jnp.int32(0)

    def chunk_body(ci, cnt):
        off = ci * _C
        pltpu.sync_copy(src_hbm.at[pl.ds(off, _C)], srcbuf)
        pltpu.sync_copy(dst_hbm.at[pl.ds(off, _C)], dstbuf)

        def scan_body(v, cnt):
            sl = pl.ds(v * 16, 16)
            dv = dstbuf[sl]
            sv = srcbuf[sl]
            msk = jnp.logical_and(dv >= base, dv < base + _R)
            csl = pl.ds(cnt, 16)
            plsc.store_compressed(cdst.at[csl], dv - base, mask=msk)
            plsc.store_compressed(csrc.at[csl], sv, mask=msk)
            plsc.store_compressed(cglb.at[csl], dv, mask=msk)
            return cnt + jnp.sum(jnp.where(msk, 1, 0))
        cnt = lax.fori_loop(0, _C // 16, scan_body, cnt)
        cnt = lax.cond(cnt >= _DRAIN_T, drain, lambda c: c, cnt)
        return cnt

    cnt = lax.fori_loop(0, _NCHUNK, chunk_body, jnp.int32(0))
    cnt = lax.cond(cnt > 0, drain, lambda c: c, cnt)

    nw = _R * _H
    pltpu.sync_copy(s_acc.at[pl.ds(0, nw)], s_out.at[pl.ds(w * nw, nw)])
    pltpu.sync_copy(mx_acc.at[pl.ds(0, nw)], mx_out.at[pl.ds(w * nw, nw)])
    pltpu.sync_copy(mn_acc.at[pl.ds(0, nw)], mn_out.at[pl.ds(w * nw, nw)])
    pltpu.sync_copy(deg_acc.at[pl.ds(0, _DEGW)],
                    deg_out.at[pl.ds(w * _DEGW, _DEGW)])


def _edge_sc(a, b, src, dst):
    mesh = plsc.VectorSubcoreMesh(core_axis_name="c", subcore_axis_name="s")
    f32 = jnp.float32
    kern = functools.partial(
        pl.kernel,
        mesh=mesh,
        compiler_params=pltpu.CompilerParams(needs_layout_passes=False),
        out_type=[
            jax.ShapeDtypeStruct((_NW * _R * _H,), f32),
            jax.ShapeDtypeStruct((_NW * _R * _H,), f32),
            jax.ShapeDtypeStruct((_NW * _R * _H,), f32),
            jax.ShapeDtypeStruct((_NW * _DEGW,), f32),
        ],
        scratch_types=[
            pltpu.VMEM((_ACCW,), f32),
            pltpu.VMEM((_ACCW,), f32),
            pltpu.VMEM((_ACCW,), f32),
            pltpu.VMEM((_DEGW,), f32),
            pltpu.VMEM((_C,), jnp.int32),
            pltpu.VMEM((_C,), jnp.int32),
            pltpu.VMEM((_CAP,), jnp.int32),
            pltpu.VMEM((_CAP,), jnp.int32),
            pltpu.VMEM((_CAP,), jnp.int32),
            pltpu.VMEM((16, _H), f32),
            pltpu.VMEM((16, _H), f32),
            pltpu.SemaphoreType.DMA,
            pltpu.SemaphoreType.DMA,
        ],
    )(_edge_body)
    return kern(a, b, src, dst)


def _tc2a_body(s_ref, mx_ref, mn_ref, deg_ref, x_ref, wu_ref, bu_ref,
               wp1_ref, bp1_ref, p1_ref, sum_ref, sq_ref):
    i = pl.program_id(0)
    deg = deg_ref[...]
    s = s_ref[...]
    mean = s / jnp.maximum(deg, 1.0)
    pos = deg > 0.0
    mx = jnp.where(pos, mx_ref[...], 0.0)
    mn = jnp.where(pos, mn_ref[...], 0.0)
    wu = wu_ref[...]
    agg = (
        jnp.dot(mean, wu[0:_H], preferred_element_type=jnp.float32)
        + jnp.dot(mx, wu[_H:2 * _H], preferred_element_type=jnp.float32)
        + jnp.dot(mn, wu[2 * _H:3 * _H], preferred_element_type=jnp.float32)
        + jnp.dot(s, wu[3 * _H:4 * _H], preferred_element_type=jnp.float32)
        + bu_ref[...]
    )
    h = jnp.maximum(agg, 0.0) + x_ref[...]
    p1 = jnp.dot(h, wp1_ref[...], preferred_element_type=jnp.float32) + bp1_ref[...]
    p1_ref[...] = p1

    @pl.when(i == 0)
    def _():
        sum_ref[...] = jnp.zeros_like(sum_ref)
        sq_ref[...] = jnp.zeros_like(sq_ref)

    sum_ref[...] += jnp.sum(p1, axis=0, keepdims=True)
    sq_ref[...] += jnp.sum(p1 * p1, axis=0, keepdims=True)


def _tc2a(s, mx, mn, deg, x, wu, bu, wp1, bp1):
    blk = 1000
    grid = _N // blk
    return pl.pallas_call(
        _tc2a_body,
        grid=(grid,),
        in_specs=[
            pl.BlockSpec((blk, _H), lambda i: (i, 0)),
            pl.BlockSpec((blk, _H), lambda i: (i, 0)),
            pl.BlockSpec((blk, _H), lambda i: (i, 0)),
            pl.BlockSpec((blk, 1), lambda i: (i, 0)),
            pl.BlockSpec((blk, _H), lambda i: (i, 0)),
            pl.BlockSpec((4 * _H, _H), lambda i: (0, 0)),
            pl.BlockSpec((1, _H), lambda i: (0, 0)),
            pl.BlockSpec((_H, _HID), lambda i: (0, 0)),
            pl.BlockSpec((1, _HID), lambda i: (0, 0)),
        ],
        out_specs=[
            pl.BlockSpec((blk, _HID), lambda i: (i, 0)),
            pl.BlockSpec((1, _HID), lambda i: (0, 0)),
            pl.BlockSpec((1, _HID), lambda i: (0, 0)),
        ],
        out_shape=[
            jax.ShapeDtypeStruct((_N, _HID), jnp.float32),
            jax.ShapeDtypeStruct((1, _HID), jnp.float32),
            jax.ShapeDtypeStruct((1, _HID), jnp.float32),
        ],
    )(s, mx, mn, deg, x, wu, bu, wp1, bp1)


def _tc2b_body(p1_ref, sum_ref, sq_ref, g_ref, be_ref, wp2_ref, bp2_ref, p_ref):
    mu = sum_ref[...] / _N
    var = sq_ref[...] / _N - mu * mu
    p1 = p1_ref[...]
    xh = (p1 - mu) * lax.rsqrt(var + 1e-5) * g_ref[...] + be_ref[...]
    xh = jnp.maximum(xh, 0.0)
    p_ref[...] = jnp.dot(xh, wp2_ref[...], preferred_element_type=jnp.float32) + bp2_ref[...]


def _tc2b(p1, psum, psq, gamma, beta, wp2, bp2):
    blk = 1000
    grid = _N // blk
    return pl.pallas_call(
        _tc2b_body,
        grid=(grid,),
        in_specs=[
            pl.BlockSpec((blk, _HID), lambda i: (i, 0)),
            pl.BlockSpec((1, _HID), lambda i: (0, 0)),
            pl.BlockSpec((1, _HID), lambda i: (0, 0)),
            pl.BlockSpec((1, _HID), lambda i: (0, 0)),
            pl.BlockSpec((1, _HID), lambda i: (0, 0)),
            pl.BlockSpec((_HID, 4), lambda i: (0, 0)),
            pl.BlockSpec((1, 4), lambda i: (0, 0)),
        ],
        out_specs=pl.BlockSpec((blk, 4), lambda i: (i, 0)),
        out_shape=jax.ShapeDtypeStruct((_N, 4), jnp.float32),
    )(p1, psum, psq, gamma, beta, wp2, bp2)


def _pair_body(p_hbm, pi_hbm, pj_hbm, out_hbm, ptab, ibuf, jbuf, obuf):
    w = lax.axis_index("s") * 2 + lax.axis_index("c")
    pbase = w * _PB
    pltpu.sync_copy(p_hbm, ptab)

    def chunk_body(ci, _):
        off = pbase + ci * _PC
        pltpu.sync_copy(pi_hbm.at[pl.ds(off, _PC)], ibuf)
        pltpu.sync_copy(pj_hbm.at[pl.ds(off, _PC)], jbuf)

        def vec_body(v, _):
            sl = pl.ds(v * 16, 16)
            iv = ibuf[sl] * 4
            jv = jbuf[sl] * 4
            a0 = plsc.load_gather(ptab, [iv])
            a1 = plsc.load_gather(ptab, [iv + 1])
            a2 = plsc.load_gather(ptab, [iv + 2])
            b0 = plsc.load_gather(ptab, [jv])
            b1 = plsc.load_gather(ptab, [jv + 1])
            b2 = plsc.load_gather(ptab, [jv + 2])
            d0 = a0 - b0
            d1 = a1 - b1
            d2 = a2 - b2
            ss = d0 * d0 + d1 * d1 + d2 * d2 + 1e-12
            ii = lax.bitcast_convert_type(ss, jnp.int32)
            yi = 0x5F3759DF - lax.shift_right_arithmetic(ii, 1)
            y = lax.bitcast_convert_type(yi, jnp.float32)
            y = y * (1.5 - 0.5 * ss * y * y)
            y = y * (1.5 - 0.5 * ss * y * y)
            y = y * (1.5 - 0.5 * ss * y * y)
            obuf[sl] = ss * y
            return 0
        lax.fori_loop(0, _PC // 16, vec_body, 0)
        pltpu.sync_copy(obuf, out_hbm.at[pl.ds(off, _PC)])
        return 0
    lax.fori_loop(0, _PB // _PC, chunk_body, 0)


def _pair_sc(p_flat, pi, pj):
    mesh = plsc.VectorSubcoreMesh(core_axis_name="c", subcore_axis_name="s")
    f32 = jnp.float32
    kern = functools.partial(
        pl.kernel,
        mesh=mesh,
        compiler_params=pltpu.CompilerParams(needs_layout_passes=False),
        out_type=jax.ShapeDtypeStruct((_PPAD,), f32),
        scratch_types=[
            pltpu.VMEM((_N * 4,), f32),
            pltpu.VMEM((_PC,), jnp.int32),
            pltpu.VMEM((_PC,), jnp.int32),
            pltpu.VMEM((_PC,), f32),
        ],
    )(_pair_body)
    return kern(p_flat, pi, pj)


def kernel(x, edge_index, pairwise_indices, W_msg, b_msg, W_upd, b_upd,
           W_p1, b_p1, gamma, beta, W_p2, b_p2):
    f32 = jnp.float32
    w1 = W_msg[:_H]
    w2 = W_msg[_H:]
    bmsg = b_msg.reshape(1, _H)

    a_tab, b_tab = _tc1(x, w1, w2, bmsg)

    src = edge_index[0]
    dst = edge_index[1]
    s_f, mx_f, mn_f, deg_f = _edge_sc(a_tab, b_tab, src, dst)

    s = s_f.reshape(_NW * _R, _H)[:_N]
    mx = mx_f.reshape(_NW * _R, _H)[:_N]
    mn = mn_f.reshape(_NW * _R, _H)[:_N]
    deg = deg_f.reshape(_NW, _DEGW)[:, :_R].reshape(_NW * _R)[:_N]
    deg = deg.reshape(_N, 1)

    p1, psum, psq = _tc2a(s, mx, mn, deg, x,
                          W_upd, b_upd.reshape(1, _H),
                          W_p1, b_p1.reshape(1, _HID))

    wp2 = jnp.concatenate([W_p2, jnp.zeros((_HID, 1), f32)], axis=1)
    bp2 = jnp.concatenate([b_p2, jnp.zeros((1,), f32)]).reshape(1, 4)
    p = _tc2b(p1, psum, psq, gamma.reshape(1, _HID), beta.reshape(1, _HID),
              wp2, bp2)

    pi = jnp.pad(pairwise_indices[0], (0, _PPAD - _P))
    pj = jnp.pad(pairwise_indices[1], (0, _PPAD - _P))
    dists = _pair_sc(p.reshape(_N * 4), pi, pj)
    return dists[:_P].reshape(_P, 1)


# speedup vs baseline: 4.8909x; 1.6767x over previous
import functools

import jax
import jax.numpy as jnp
from jax import lax
from jax.experimental import pallas as pl
from jax.experimental.pallas import tpu as pltpu
from jax.experimental.pallas import tpu_sc as plsc

_N = 10000
_E = 320000
_P = 500000
_H = 128
_HID = 32

_NW = 32
_R = 313
_RROWS = _R + 1
_ACCW = _RROWS * _H
_DEGW = 336
_C = 512
_NCHUNK = _E // _C
_CAP = 768
_DRAIN_T = 240
_NEG = -3.0e38
_POS = 3.0e38

_PB = 16384
_PPAD = _PB * _NW
_PC = 1024


def _tc1_body(x_ref, w1_ref, w2_ref, b_ref, a_ref, b_out_ref):
    xv = x_ref[...]
    a_ref[...] = jnp.dot(xv, w1_ref[...], preferred_element_type=jnp.float32)
    b_out_ref[...] = (
        jnp.dot(xv, w2_ref[...], preferred_element_type=jnp.float32) + b_ref[...]
    )


def _tc1(x, w1, w2, bmsg):
    blk = 1000
    grid = _N // blk
    return pl.pallas_call(
        _tc1_body,
        grid=(grid,),
        in_specs=[
            pl.BlockSpec((blk, _H), lambda i: (i, 0)),
            pl.BlockSpec((_H, _H), lambda i: (0, 0)),
            pl.BlockSpec((_H, _H), lambda i: (0, 0)),
            pl.BlockSpec((1, _H), lambda i: (0, 0)),
        ],
        out_specs=[
            pl.BlockSpec((blk, _H), lambda i: (i, 0)),
            pl.BlockSpec((blk, _H), lambda i: (i, 0)),
        ],
        out_shape=[
            jax.ShapeDtypeStruct((_N, _H), jnp.float32),
            jax.ShapeDtypeStruct((_N, _H), jnp.float32),
        ],
    )(x, w1, w2, bmsg)


def _edge_body(a_hbm, b_hbm, epk_hbm,
               s_out, mx_out, mn_out, deg_out,
               s_acc, mx_acc, mn_acc, deg_acc,
               ebuf, clist,
               stage_a, stage_b, sem_c, sem_a, sem_b):
    w = lax.axis_index("s") * 2 + lax.axis_index("c")
    base = w * _R
    iota = lax.iota(jnp.int32, 16)
    onehot = jnp.where(iota == 0, 1.0, 0.0).astype(jnp.float32)
    zero16f = jnp.zeros((16,), jnp.float32)

    def init_body(i, _):
        sl = pl.ds(i * 16, 16)
        s_acc[sl] = zero16f
        mx_acc[sl] = jnp.full((16,), _NEG, jnp.float32)
        mn_acc[sl] = jnp.full((16,), _POS, jnp.float32)
        return 0
    lax.fori_loop(0, _ACCW // 16, init_body, 0)

    def initd_body(i, _):
        deg_acc[pl.ds(i * 16, 16)] = zero16f
        return 0
    lax.fori_loop(0, _DEGW // 16, initd_body, 0)

    def _unpack(j):
        ev = clist[pl.ds(j * 16, 16)]
        sv = lax.shift_right_logical(ev, 14)
        dv = jnp.bitwise_and(ev, 16383)
        return sv, dv

    def _issue_gather(j):
        p = lax.rem(j, 2)
        sv, dv = _unpack(j)
        pltpu.async_copy(a_hbm.at[sv], stage_a.at[p], sem_a)
        pltpu.async_copy(b_hbm.at[dv], stage_b.at[p], sem_b)

    def drain(cnt):
        clist[pl.ds(cnt, 16)] = jnp.full((16,), base + _R, jnp.int32)
        nvec = (cnt + 15) // 16
        _issue_gather(0)

        def gbody(j, _):
            p = lax.rem(j, 2)
            sv, dv = _unpack(j)
            pltpu.make_async_copy(a_hbm.at[sv], stage_a.at[p], sem_a).wait()
            pltpu.make_async_copy(b_hbm.at[dv], stage_b.at[p], sem_b).wait()

            @pl.when(j + 1 < nvec)
            def _():
                _issue_gather(j + 1)

            dlv = dv - base

            def edge_body(k, _):
                dk = jnp.sum(jnp.where(iota == k, dlv, 0))
                bofs = dk * _H
                for r in range(_H // 16):
                    asl = stage_a.at[p].at[k][pl.ds(r * 16, 16)]
                    bsl = stage_b.at[p].at[k][pl.ds(r * 16, 16)]
                    m = jnp.maximum(asl + bsl, 0.0)
                    accsl = pl.ds(bofs + r * 16, 16)
                    plsc.addupdate(s_acc.at[accsl], m)
                    mx_acc[accsl] = jnp.maximum(mx_acc[accsl], m)
                    mn_acc[accsl] = jnp.minimum(mn_acc[accsl], m)
                plsc.addupdate(deg_acc.at[pl.ds(dk, 16)], onehot)
                return 0
            lax.fori_loop(0, 16, edge_body, 0)
            return 0
        lax.fori_loop(0, nvec, gbody, 0)
        return jnp.int32(0)

    def _issue_chunk(ci):
        p = lax.rem(ci, 2)
        pltpu.async_copy(epk_hbm.at[pl.ds(ci * _C, _C)],
                         ebuf.at[pl.ds(p * _C, _C)], sem_c)

    _issue_chunk(0)

    def chunk_body(ci, cnt):
        p = lax.rem(ci, 2)
        pltpu.make_async_copy(epk_hbm.at[pl.ds(ci * _C, _C)],
                              ebuf.at[pl.ds(p * _C, _C)], sem_c).wait()

        @pl.when(ci + 1 < _NCHUNK)
        def _():
            _issue_chunk(ci + 1)

        pbase_w = p * _C

        def scan_body(v, cnt):
            ev = ebuf[pl.ds(pbase_w + v * 16, 16)]
            dv = jnp.bitwise_and(ev, 16383)
            msk = jnp.logical_and(dv >= base, dv < base + _R)
            plsc.store_compressed(clist.at[pl.ds(cnt, 16)], ev, mask=msk)
            return cnt + plsc.all_reduce_population_count(msk)[0]
        cnt = lax.fori_loop(0, _C // 16, scan_body, cnt)
        cnt = lax.cond(cnt >= _DRAIN_T, drain, lambda c: c, cnt)
        return cnt

    cnt = lax.fori_loop(0, _NCHUNK, chunk_body, jnp.int32(0))
    cnt = lax.cond(cnt > 0, drain, lambda c: c, cnt)

    nw = _R * _H
    pltpu.sync_copy(s_acc.at[pl.ds(0, nw)], s_out.at[pl.ds(w * nw, nw)])
    pltpu.sync_copy(mx_acc.at[pl.ds(0, nw)], mx_out.at[pl.ds(w * nw, nw)])
    pltpu.sync_copy(mn_acc.at[pl.ds(0, nw)], mn_out.at[pl.ds(w * nw, nw)])
    pltpu.sync_copy(deg_acc.at[pl.ds(0, _DEGW)],
                    deg_out.at[pl.ds(w * _DEGW, _DEGW)])


def _edge_sc(a, b, epk):
    mesh = plsc.VectorSubcoreMesh(core_axis_name="c", subcore_axis_name="s")
    f32 = jnp.float32
    kern = functools.partial(
        pl.kernel,
        mesh=mesh,
        compiler_params=pltpu.CompilerParams(needs_layout_passes=False),
        out_type=[
            jax.ShapeDtypeStruct((_NW * _R * _H,), f32),
            jax.ShapeDtypeStruct((_NW * _R * _H,), f32),
            jax.ShapeDtypeStruct((_NW * _R * _H,), f32),
            jax.ShapeDtypeStruct((_NW * _DEGW,), f32),
        ],
        scratch_types=[
            pltpu.VMEM((_ACCW,), f32),
            pltpu.VMEM((_ACCW,), f32),
            pltpu.VMEM((_ACCW,), f32),
            pltpu.VMEM((_DEGW,), f32),
            pltpu.VMEM((2 * _C,), jnp.int32),
            pltpu.VMEM((_CAP,), jnp.int32),
            pltpu.VMEM((2, 16, _H), f32),
            pltpu.VMEM((2, 16, _H), f32),
            pltpu.SemaphoreType.DMA,
            pltpu.SemaphoreType.DMA,
            pltpu.SemaphoreType.DMA,
        ],
    )(_edge_body)
    return kern(a, b, epk)


def _tc2a_body(s_ref, mx_ref, mn_ref, deg_ref, x_ref, wu_ref, bu_ref,
               wp1_ref, bp1_ref, p1_ref, sum_ref, sq_ref):
    i = pl.program_id(0)
    deg = deg_ref[...]
    s = s_ref[...]
    mean = s / jnp.maximum(deg, 1.0)
    pos = deg > 0.0
    mx = jnp.where(pos, mx_ref[...], 0.0)
    mn = jnp.where(pos, mn_ref[...], 0.0)
    wu = wu_ref[...]
    agg = (
        jnp.dot(mean, wu[0:_H], preferred_element_type=jnp.float32)
        + jnp.dot(mx, wu[_H:2 * _H], preferred_element_type=jnp.float32)
        + jnp.dot(mn, wu[2 * _H:3 * _H], preferred_element_type=jnp.float32)
        + jnp.dot(s, wu[3 * _H:4 * _H], preferred_element_type=jnp.float32)
        + bu_ref[...]
    )
    h = jnp.maximum(agg, 0.0) + x_ref[...]
    p1 = jnp.dot(h, wp1_ref[...], preferred_element_type=jnp.float32) + bp1_ref[...]
    p1_ref[...] = p1

    @pl.when(i == 0)
    def _():
        sum_ref[...] = jnp.zeros_like(sum_ref)
        sq_ref[...] = jnp.zeros_like(sq_ref)

    sum_ref[...] += jnp.sum(p1, axis=0, keepdims=True)
    sq_ref[...] += jnp.sum(p1 * p1, axis=0, keepdims=True)


def _tc2a(s, mx, mn, deg, x, wu, bu, wp1, bp1):
    blk = 1000
    grid = _N // blk
    return pl.pallas_call(
        _tc2a_body,
        grid=(grid,),
        in_specs=[
            pl.BlockSpec((blk, _H), lambda i: (i, 0)),
            pl.BlockSpec((blk, _H), lambda i: (i, 0)),
            pl.BlockSpec((blk, _H), lambda i: (i, 0)),
            pl.BlockSpec((blk, 1), lambda i: (i, 0)),
            pl.BlockSpec((blk, _H), lambda i: (i, 0)),
            pl.BlockSpec((4 * _H, _H), lambda i: (0, 0)),
            pl.BlockSpec((1, _H), lambda i: (0, 0)),
            pl.BlockSpec((_H, _HID), lambda i: (0, 0)),
            pl.BlockSpec((1, _HID), lambda i: (0, 0)),
        ],
        out_specs=[
            pl.BlockSpec((blk, _HID), lambda i: (i, 0)),
            pl.BlockSpec((1, _HID), lambda i: (0, 0)),
            pl.BlockSpec((1, _HID), lambda i: (0, 0)),
        ],
        out_shape=[
            jax.ShapeDtypeStruct((_N, _HID), jnp.float32),
            jax.ShapeDtypeStruct((1, _HID), jnp.float32),
            jax.ShapeDtypeStruct((1, _HID), jnp.float32),
        ],
    )(s, mx, mn, deg, x, wu, bu, wp1, bp1)


def _tc2b_body(p1_ref, sum_ref, sq_ref, g_ref, be_ref, wp2_ref, bp2_ref, p_ref):
    mu = sum_ref[...] / _N
    var = sq_ref[...] / _N - mu * mu
    p1 = p1_ref[...]
    xh = (p1 - mu) * lax.rsqrt(var + 1e-5) * g_ref[...] + be_ref[...]
    xh = jnp.maximum(xh, 0.0)
    p_ref[...] = jnp.dot(xh, wp2_ref[...], preferred_element_type=jnp.float32) + bp2_ref[...]


def _tc2b(p1, psum, psq, gamma, beta, wp2, bp2):
    blk = 1000
    grid = _N // blk
    return pl.pallas_call(
        _tc2b_body,
        grid=(grid,),
        in_specs=[
            pl.BlockSpec((blk, _HID), lambda i: (i, 0)),
            pl.BlockSpec((1, _HID), lambda i: (0, 0)),
            pl.BlockSpec((1, _HID), lambda i: (0, 0)),
            pl.BlockSpec((1, _HID), lambda i: (0, 0)),
            pl.BlockSpec((1, _HID), lambda i: (0, 0)),
            pl.BlockSpec((_HID, 4), lambda i: (0, 0)),
            pl.BlockSpec((1, 4), lambda i: (0, 0)),
        ],
        out_specs=pl.BlockSpec((blk, 4), lambda i: (i, 0)),
        out_shape=jax.ShapeDtypeStruct((_N, 4), jnp.float32),
    )(p1, psum, psq, gamma, beta, wp2, bp2)


def _pair_body(p_hbm, pi_hbm, pj_hbm, out_hbm, ptab, ibuf, jbuf, obuf):
    w = lax.axis_index("s") * 2 + lax.axis_index("c")
    pbase = w * _PB
    pltpu.sync_copy(p_hbm, ptab)

    def chunk_body(ci, _):
        off = pbase + ci * _PC
        pltpu.sync_copy(pi_hbm.at[pl.ds(off, _PC)], ibuf)
        pltpu.sync_copy(pj_hbm.at[pl.ds(off, _PC)], jbuf)

        def vec_body(v, _):
            sl = pl.ds(v * 16, 16)
            iv = ibuf[sl] * 4
            jv = jbuf[sl] * 4
            a0 = plsc.load_gather(ptab, [iv])
            a1 = plsc.load_gather(ptab, [iv + 1])
            a2 = plsc.load_gather(ptab, [iv + 2])
            b0 = plsc.load_gather(ptab, [jv])
            b1 = plsc.load_gather(ptab, [jv + 1])
            b2 = plsc.load_gather(ptab, [jv + 2])
            d0 = a0 - b0
            d1 = a1 - b1
            d2 = a2 - b2
            ss = d0 * d0 + d1 * d1 + d2 * d2 + 1e-12
            ii = lax.bitcast_convert_type(ss, jnp.int32)
            yi = 0x5F3759DF - lax.shift_right_arithmetic(ii, 1)
            y = lax.bitcast_convert_type(yi, jnp.float32)
            y = y * (1.5 - 0.5 * ss * y * y)
            y = y * (1.5 - 0.5 * ss * y * y)
            y = y * (1.5 - 0.5 * ss * y * y)
            obuf[sl] = ss * y
            return 0
        lax.fori_loop(0, _PC // 16, vec_body, 0)
        pltpu.sync_copy(obuf, out_hbm.at[pl.ds(off, _PC)])
        return 0
    lax.fori_loop(0, _PB // _PC, chunk_body, 0)


def _pair_sc(p_flat, pi, pj):
    mesh = plsc.VectorSubcoreMesh(core_axis_name="c", subcore_axis_name="s")
    f32 = jnp.float32
    kern = functools.partial(
        pl.kernel,
        mesh=mesh,
        compiler_params=pltpu.CompilerParams(needs_layout_passes=False),
        out_type=jax.ShapeDtypeStruct((_PPAD,), f32),
        scratch_types=[
            pltpu.VMEM((_N * 4,), f32),
            pltpu.VMEM((_PC,), jnp.int32),
            pltpu.VMEM((_PC,), jnp.int32),
            pltpu.VMEM((_PC,), f32),
        ],
    )(_pair_body)
    return kern(p_flat, pi, pj)


def kernel(x, edge_index, pairwise_indices, W_msg, b_msg, W_upd, b_upd,
           W_p1, b_p1, gamma, beta, W_p2, b_p2):
    f32 = jnp.float32
    w1 = W_msg[:_H]
    w2 = W_msg[_H:]
    bmsg = b_msg.reshape(1, _H)

    a_tab, b_tab = _tc1(x, w1, w2, bmsg)
    b_pad = jnp.pad(b_tab, ((0, _NW * _R + 16 - _N), (0, 0)))

    epk = edge_index[0] * 16384 + edge_index[1]
    s_f, mx_f, mn_f, deg_f = _edge_sc(a_tab, b_pad, epk)

    s = s_f.reshape(_NW * _R, _H)[:_N]
    mx = mx_f.reshape(_NW * _R, _H)[:_N]
    mn = mn_f.reshape(_NW * _R, _H)[:_N]
    deg = deg_f.reshape(_NW, _DEGW)[:, :_R].reshape(_NW * _R)[:_N]
    deg = deg.reshape(_N, 1)

    p1, psum, psq = _tc2a(s, mx, mn, deg, x,
                          W_upd, b_upd.reshape(1, _H),
                          W_p1, b_p1.reshape(1, _HID))

    wp2 = jnp.concatenate([W_p2, jnp.zeros((_HID, 1), f32)], axis=1)
    bp2 = jnp.concatenate([b_p2, jnp.zeros((1,), f32)]).reshape(1, 4)
    p = _tc2b(p1, psum, psq, gamma.reshape(1, _HID), beta.reshape(1, _HID),
              wp2, bp2)

    pi = jnp.pad(pairwise_indices[0], (0, _PPAD - _P))
    pj = jnp.pad(pairwise_indices[1], (0, _PPAD - _P))
    dists = _pair_sc(p.reshape(_N * 4), pi, pj)
    return dists[:_P].reshape(_P, 1)
